# baseline (speedup 1.0000x reference)
import functools

import jax
import jax.numpy as jnp
from jax import lax
from jax.experimental import pallas as pl
from jax.experimental.pallas import tpu as pltpu
from jax.experimental.pallas import tpu_sc as plsc

_INT_MIN = -(2 ** 31)


def _sc_sort_rows(vt, xors16, C):
    R, S = vt.shape
    G = 8
    NC, NS = 2, 16
    NW = NC * NS
    TASKS = R // G
    TPW = TASKS // NW
    VR = S // 16
    VSH = VR.bit_length() - 1
    mesh = plsc.VectorSubcoreMesh(core_axis_name="c", subcore_axis_name="s")

    @functools.partial(
        pl.kernel,
        out_type=jax.ShapeDtypeStruct((R, S), jnp.float32),
        mesh=mesh,
        scratch_types=[
            pltpu.VMEM((G, S), jnp.float32),
            pltpu.VMEM((S,), jnp.int32),
            pltpu.VMEM((S,), jnp.int32),
            pltpu.VMEM((4096,), jnp.int32),
            pltpu.VMEM((C * 16,), jnp.int32),
        ],
        compiler_params=pltpu.CompilerParams(needs_layout_passes=False),
    )
    def run(v_hbm, xors_hbm, out_hbm, fbuf, ka, kb, hist, xor_v):
        lane = lax.iota(jnp.int32, 16)
        ones = jnp.ones((16,), jnp.int32)
        wid = lax.axis_index("s") * NC + lax.axis_index("c")
        pltpu.sync_copy(xors_hbm, xor_v)

        def radix_pass(src, dst, sh):
            def zero(d, _):
                hist[pl.ds(d * 16, 16)] = jnp.zeros((16,), jnp.int32)
                return 0
            lax.fori_loop(0, 256, zero, 0)

            def count(i, _):
                kk = src[pl.ds(i * 16, 16)]
                d = lax.shift_right_logical(kk, sh) & 255
                plsc.addupdate_scatter(hist, [(d << 4) | lane], ones)
                return 0
            lax.fori_loop(0, VR, count, 0)

            def scan(d, run_):
                r = hist[pl.ds(d * 16, 16)]
                cum = plsc.cumsum(r)
                tot = jnp.sum(r)
                hist[pl.ds(d * 16, 16)] = (cum - r) + run_
                return run_ + tot
            lax.fori_loop(0, 256, scan, jnp.int32(0))

            def perm(i, _):
                kk = src[pl.ds(i * 16, 16)]
                d = lax.shift_right_logical(kk, sh) & 255
                slot = (d << 4) | lane
                pos = plsc.load_gather(hist, [slot])
                plsc.addupdate_scatter(hist, [slot], ones)
                addr = ((pos & (VR - 1)) << 4) | lax.shift_right_logical(pos, VSH)
                plsc.store_scatter(dst, [addr], kk)
                return 0
            lax.fori_loop(0, VR, perm, 0)

        def task_body(t, _):
            r0 = (wid * TPW + t) * G
            pltpu.sync_copy(v_hbm.at[pl.ds(r0, G), :], fbuf)

            def col_body(cl, _):
                c = (r0 % C) + cl
                xvec = xor_v[pl.ds(c * 16, 16)]

                def tin(i, _):
                    x = fbuf[cl, pl.ds(i * 16, 16)]
                    b = lax.bitcast_convert_type(x, jnp.int32)
                    m = lax.shift_right_arithmetic(b, 31)
                    ka[pl.ds(i * 16, 16)] = b ^ (m | _INT_MIN) ^ xvec
                    return 0
                lax.fori_loop(0, VR, tin, 0)

                radix_pass(ka, kb, 0)
                radix_pass(kb, ka, 8)
                radix_pass(ka, kb, 16)
                radix_pass(kb, ka, 24)

                def tout(i, _):
                    kk = ka[pl.ds(i * 16, 16)]
                    t2 = kk ^ xvec
                    bb = jnp.where(t2 < 0, t2 ^ _INT_MIN, ~t2)
                    x = lax.bitcast_convert_type(bb, jnp.float32)
                    plsc.store_scatter(
                        fbuf, [jnp.full((16,), cl, jnp.int32), lane * VR + i], x)
                    return 0
                lax.fori_loop(0, VR, tout, 0)
                return 0
            lax.fori_loop(0, G, col_body, 0)
            pltpu.sync_copy(fbuf, out_hbm.at[pl.ds(r0, G), :])
            return 0
        lax.fori_loop(0, TPW, task_body, 0)

    return run(vt, xors16)


def kernel(q, k, v, col_descend):
    B, H, S, C = v.shape
    mask = jnp.zeros((C,), jnp.bool_).at[col_descend.reshape(-1)].set(True)
    xors = jnp.where(mask, jnp.int32(-1), jnp.int32(0))
    xors16 = jnp.broadcast_to(xors.reshape(C, 1), (C, 16)).reshape(C * 16)
    vt = v.reshape(B * H, S, C).swapaxes(1, 2).reshape(B * H * C, S)
    out_t = _sc_sort_rows(vt, xors16, C)
    return (out_t.reshape(B * H, C, S).swapaxes(1, 2)
            .reshape(B, H, S, C))


# speedup vs baseline: 1.8749x; 1.8749x over previous
import functools

import jax
import jax.numpy as jnp
from jax import lax
from jax.experimental import pallas as pl
from jax.experimental.pallas import tpu as pltpu
from jax.experimental.pallas import tpu_sc as plsc

_INT_MIN = -(2 ** 31)
_NCOLS = 4


def _sc_sort_rows(vt, xors16, C):
    R, S = vt.shape
    G = 8
    NC, NS = 2, 16
    NW = NC * NS
    TASKS = R // G
    TPW = TASKS // NW
    VR = S // 16
    VSH = VR.bit_length() - 1
    mesh = plsc.VectorSubcoreMesh(core_axis_name="c", subcore_axis_name="s")

    @functools.partial(
        pl.kernel,
        out_type=jax.ShapeDtypeStruct((R, S), jnp.float32),
        mesh=mesh,
        scratch_types=[
            pltpu.VMEM((G, S), jnp.float32),
            [pltpu.VMEM((S,), jnp.int32) for _ in range(_NCOLS)],
            [pltpu.VMEM((4096,), jnp.int32) for _ in range(_NCOLS)],
            pltpu.VMEM((C * 16,), jnp.int32),
        ],
        compiler_params=pltpu.CompilerParams(needs_layout_passes=False),
    )
    def run(v_hbm, xors_hbm, out_hbm, fbuf, kbufs, hists, xor_v):
        lane = lax.iota(jnp.int32, 16)
        ones = jnp.ones((16,), jnp.int32)
        wid = lax.axis_index("s") * NC + lax.axis_index("c")
        pltpu.sync_copy(xors_hbm, xor_v)

        def process_group(r0, cols):
            xvecs = [xor_v[pl.ds(((r0 % C) + cl) * 16, 16)] for cl in cols]
            fulls = [jnp.full((16,), cl, jnp.int32) for cl in cols]

            def src_load(p, j, i):
                if p == 0:
                    x = fbuf[cols[j], pl.ds(i * 16, 16)]
                    b = lax.bitcast_convert_type(x, jnp.int32)
                    m = lax.shift_right_arithmetic(b, 31)
                    return b ^ (m | _INT_MIN) ^ xvecs[j]
                if p % 2 == 1:
                    return kbufs[j][pl.ds(i * 16, 16)]
                x = fbuf[cols[j], pl.ds(i * 16, 16)]
                return lax.bitcast_convert_type(x, jnp.int32)

            def dst_store(p, j, pos, kk):
                if p == 3:
                    t2 = kk ^ xvecs[j]
                    bb = jnp.where(t2 < 0, t2 ^ _INT_MIN, ~t2)
                    x = lax.bitcast_convert_type(bb, jnp.float32)
                    plsc.store_scatter(fbuf, [fulls[j], pos], x)
                    return
                addr = ((pos & (VR - 1)) << 4) | lax.shift_right_logical(pos, VSH)
                if p % 2 == 0:
                    plsc.store_scatter(kbufs[j], [addr], kk)
                else:
                    x = lax.bitcast_convert_type(kk, jnp.float32)
                    plsc.store_scatter(fbuf, [fulls[j], addr], x)

            for p, sh in enumerate((0, 8, 16, 24)):
                @plsc.parallel_loop(0, 256, 1, unroll=4)
                def _zero(d):
                    for j in range(_NCOLS):
                        hists[j][pl.ds(d * 16, 16)] = jnp.zeros((16,), jnp.int32)

                @plsc.parallel_loop(0, VR, 1, unroll=2)
                def _count(i):
                    for j in range(_NCOLS):
                        kk = src_load(p, j, i)
                        d = lax.shift_right_logical(kk, sh) & 255
                        plsc.addupdate_scatter(
                            hists[j], [(d << 4) | lane], ones)

                def _scan(d, runs):
                    out = []
                    for j in range(_NCOLS):
                        r = hists[j][pl.ds(d * 16, 16)]
                        cum = plsc.cumsum(r)
                        tot = jnp.sum(r)
                        hists[j][pl.ds(d * 16, 16)] = (cum - r) + runs[j]
                        out.append(runs[j] + tot)
                    return tuple(out)
                lax.fori_loop(0, 256, _scan, (jnp.int32(0),) * _NCOLS)

                def _perm(i, _):
                    for j in range(_NCOLS):
                        kk = src_load(p, j, i)
                        d = lax.shift_right_logical(kk, sh) & 255
                        slot = (d << 4) | lane
                        pos = plsc.load_gather(hists[j], [slot])
                        plsc.addupdate_scatter(hists[j], [slot], ones)
                        dst_store(p, j, pos, kk)
                    return 0
                lax.fori_loop(0, VR, _perm, 0)

        def task_body(t, _):
            r0 = (wid * TPW + t) * G
            pltpu.sync_copy(v_hbm.at[pl.ds(r0, G), :], fbuf)
            for h in range(G // _NCOLS):
                process_group(r0, [h * _NCOLS + j for j in range(_NCOLS)])
            pltpu.sync_copy(fbuf, out_hbm.at[pl.ds(r0, G), :])
            return 0
        lax.fori_loop(0, TPW, task_body, 0)

    return run(vt, xors16)


def kernel(q, k, v, col_descend):
    B, H, S, C = v.shape
    mask = jnp.zeros((C,), jnp.bool_).at[col_descend.reshape(-1)].set(True)
    xors = jnp.where(mask, jnp.int32(-1), jnp.int32(0))
    xors16 = jnp.broadcast_to(xors.reshape(C, 1), (C, 16)).reshape(C * 16)
    vt = v.reshape(B * H, S, C).swapaxes(1, 2).reshape(B * H * C, S)
    out_t = _sc_sort_rows(vt, xors16, C)
    return (out_t.reshape(B * H, C, S).swapaxes(1, 2)
            .reshape(B, H, S, C))


# speedup vs baseline: 1.8904x; 1.0083x over previous
import functools

import jax
import jax.numpy as jnp
from jax import lax
from jax.experimental import pallas as pl
from jax.experimental.pallas import tpu as pltpu
from jax.experimental.pallas import tpu_sc as plsc

_INT_MIN = -(2 ** 31)
_NCOLS = 4


def _sc_sort_rows(vt, xors16, C):
    R, S = vt.shape
    G = 8
    NC, NS = 2, 16
    NW = NC * NS
    TASKS = R // G
    TPW = TASKS // NW
    VR = S // 16
    VSH = VR.bit_length() - 1
    mesh = plsc.VectorSubcoreMesh(core_axis_name="c", subcore_axis_name="s")

    @functools.partial(
        pl.kernel,
        out_type=jax.ShapeDtypeStruct((R, S), jnp.float32),
        mesh=mesh,
        scratch_types=[
            pltpu.VMEM((G, S), jnp.float32),
            [pltpu.VMEM((S,), jnp.int32) for _ in range(_NCOLS)],
            [pltpu.VMEM((4096,), jnp.int32) for _ in range(_NCOLS)],
            pltpu.VMEM((C * 16,), jnp.int32),
        ],
        compiler_params=pltpu.CompilerParams(needs_layout_passes=False),
    )
    def run(v_hbm, xors_hbm, out_hbm, fbuf, kbufs, hists, xor_v):
        lane = lax.iota(jnp.int32, 16)
        ones = jnp.ones((16,), jnp.int32)
        wid = lax.axis_index("s") * NC + lax.axis_index("c")
        pltpu.sync_copy(xors_hbm, xor_v)

        def process_group(r0, cols):
            xvecs = [xor_v[pl.ds(((r0 % C) + cl) * 16, 16)] for cl in cols]
            fulls = [jnp.full((16,), cl, jnp.int32) for cl in cols]

            def src_load(p, j, i):
                if p == 0:
                    x = fbuf[cols[j], pl.ds(i * 16, 16)]
                    b = lax.bitcast_convert_type(x, jnp.int32)
                    m = lax.shift_right_arithmetic(b, 31)
                    return b ^ (m | _INT_MIN) ^ xvecs[j]
                if p % 2 == 1:
                    return kbufs[j][pl.ds(i * 16, 16)]
                x = fbuf[cols[j], pl.ds(i * 16, 16)]
                return lax.bitcast_convert_type(x, jnp.int32)

            def dst_store(p, j, pos, kk):
                if p == 3:
                    t2 = kk ^ xvecs[j]
                    bb = jnp.where(t2 < 0, t2 ^ _INT_MIN, ~t2)
                    x = lax.bitcast_convert_type(bb, jnp.float32)
                    plsc.store_scatter(fbuf, [fulls[j], pos], x)
                    return
                addr = ((pos & (VR - 1)) << 4) | lax.shift_right_logical(pos, VSH)
                if p % 2 == 0:
                    plsc.store_scatter(kbufs[j], [addr], kk)
                else:
                    x = lax.bitcast_convert_type(kk, jnp.float32)
                    plsc.store_scatter(fbuf, [fulls[j], addr], x)

            for p, sh in enumerate((0, 8, 16, 24)):
                @plsc.parallel_loop(0, 256, 1, unroll=4)
                def _zero(d):
                    for j in range(_NCOLS):
                        hists[j][pl.ds(d * 16, 16)] = jnp.zeros((16,), jnp.int32)

                @plsc.parallel_loop(0, VR, 1, unroll=2)
                def _count(i):
                    for j in range(_NCOLS):
                        kk = src_load(p, j, i)
                        d = lax.shift_right_logical(kk, sh) & 255
                        plsc.addupdate_scatter(
                            hists[j], [(d << 4) | lane], ones)

                def _scan(d, runs):
                    out = []
                    for j in range(_NCOLS):
                        r = hists[j][pl.ds(d * 16, 16)]
                        cum = plsc.cumsum(r)
                        tot = jnp.sum(r)
                        hists[j][pl.ds(d * 16, 16)] = (cum - r) + runs[j]
                        out.append(runs[j] + tot)
                    return tuple(out)
                lax.fori_loop(0, 256, _scan, (jnp.int32(0),) * _NCOLS)

                def _perm(i2, _):
                    for u in range(2):
                        i = i2 * 2 + u
                        for j in range(_NCOLS):
                            kk = src_load(p, j, i)
                            d = lax.shift_right_logical(kk, sh) & 255
                            slot = (d << 4) | lane
                            pos = plsc.load_gather(hists[j], [slot])
                            plsc.addupdate_scatter(hists[j], [slot], ones)
                            dst_store(p, j, pos, kk)
                    return 0
                lax.fori_loop(0, VR // 2, _perm, 0)

        def task_body(t, _):
            r0 = (wid * TPW + t) * G
            pltpu.sync_copy(v_hbm.at[pl.ds(r0, G), :], fbuf)
            for h in range(G // _NCOLS):
                process_group(r0, [h * _NCOLS + j for j in range(_NCOLS)])
            pltpu.sync_copy(fbuf, out_hbm.at[pl.ds(r0, G), :])
            return 0
        lax.fori_loop(0, TPW, task_body, 0)

    return run(vt, xors16)


def kernel(q, k, v, col_descend):
    B, H, S, C = v.shape
    mask = jnp.zeros((C,), jnp.bool_).at[col_descend.reshape(-1)].set(True)
    xors = jnp.where(mask, jnp.int32(-1), jnp.int32(0))
    xors16 = jnp.broadcast_to(xors.reshape(C, 1), (C, 16)).reshape(C * 16)
    vt = v.reshape(B * H, S, C).swapaxes(1, 2).reshape(B * H * C, S)
    out_t = _sc_sort_rows(vt, xors16, C)
    return (out_t.reshape(B * H, C, S).swapaxes(1, 2)
            .reshape(B, H, S, C))


# speedup vs baseline: 3.1182x; 1.6495x over previous
import functools

import jax
import jax.numpy as jnp
from jax import lax
from jax.experimental import pallas as pl
from jax.experimental.pallas import tpu as pltpu
from jax.experimental.pallas import tpu_sc as plsc

_INT_MIN = -(2 ** 31)
_NCOLS = 4


def _sc_sort_rows(vt, xors16, C):
    R, S = vt.shape
    G = 8
    NC, NS = 2, 16
    NW = NC * NS
    TASKS = R // G
    TPW = TASKS // NW
    VR = S // 16
    VSH = VR.bit_length() - 1
    mesh = plsc.VectorSubcoreMesh(core_axis_name="c", subcore_axis_name="s")

    @functools.partial(
        pl.kernel,
        out_type=jax.ShapeDtypeStruct((R, S), jnp.float32),
        mesh=mesh,
        scratch_types=[
            pltpu.VMEM((G, S), jnp.float32),
            [pltpu.VMEM((S,), jnp.int32) for _ in range(_NCOLS)],
            [pltpu.VMEM((4096,), jnp.int32) for _ in range(_NCOLS)],
            pltpu.VMEM((C * 16,), jnp.int32),
        ],
        compiler_params=pltpu.CompilerParams(needs_layout_passes=False),
    )
    def run(v_hbm, xors_hbm, out_hbm, fbuf, kbufs, hists, xor_v):
        lane = lax.iota(jnp.int32, 16)
        ones = jnp.ones((16,), jnp.int32)
        wid = lax.axis_index("s") * NC + lax.axis_index("c")
        pltpu.sync_copy(xors_hbm, xor_v)

        def process_group(r0, cols):
            xvecs = [xor_v[pl.ds(((r0 % C) + cl) * 16, 16)] for cl in cols]
            fulls = [jnp.full((16,), cl, jnp.int32) for cl in cols]

            def src_load(p, j, i):
                if p == 0:
                    x = fbuf[cols[j], pl.ds(i * 16, 16)]
                    b = lax.bitcast_convert_type(x, jnp.int32)
                    m = lax.shift_right_arithmetic(b, 31)
                    return b ^ (m | _INT_MIN) ^ xvecs[j]
                if p % 2 == 1:
                    return kbufs[j][pl.ds(i * 16, 16)]
                x = fbuf[cols[j], pl.ds(i * 16, 16)]
                return lax.bitcast_convert_type(x, jnp.int32)

            def dst_store(p, j, pos, kk):
                if p == 3:
                    t2 = kk ^ xvecs[j]
                    bb = jnp.where(t2 < 0, t2 ^ _INT_MIN, ~t2)
                    x = lax.bitcast_convert_type(bb, jnp.float32)
                    plsc.store_scatter(fbuf, [fulls[j], pos], x)
                    return
                addr = ((pos & (VR - 1)) << 4) | lax.shift_right_logical(pos, VSH)
                if p % 2 == 0:
                    plsc.store_scatter(kbufs[j], [addr], kk)
                else:
                    x = lax.bitcast_convert_type(kk, jnp.float32)
                    plsc.store_scatter(fbuf, [fulls[j], addr], x)

            for p, sh in enumerate((0, 8, 16, 24)):
                @plsc.parallel_loop(0, 256, 1, unroll=4)
                def _zero(d):
                    for j in range(_NCOLS):
                        hists[j][pl.ds(d * 16, 16)] = jnp.zeros((16,), jnp.int32)

                @plsc.parallel_loop(0, VR, 1, unroll=2)
                def _count(i):
                    for j in range(_NCOLS):
                        kk = src_load(p, j, i)
                        d = lax.shift_right_logical(kk, sh) & 255
                        plsc.addupdate_scatter(
                            hists[j], [(d << 4) | lane], ones)

                def _scan(d, runs):
                    out = []
                    for j in range(_NCOLS):
                        r = hists[j][pl.ds(d * 16, 16)]
                        cum = plsc.cumsum(r)
                        tot = jnp.sum(r)
                        hists[j][pl.ds(d * 16, 16)] = (cum - r) + runs[j]
                        out.append(runs[j] + tot)
                    return tuple(out)
                lax.fori_loop(0, 256, _scan, (jnp.int32(0),) * _NCOLS)

                def _perm(i2, _):
                    for u in range(2):
                        i = i2 * 2 + u
                        for j in range(_NCOLS):
                            kk = src_load(p, j, i)
                            d = lax.shift_right_logical(kk, sh) & 255
                            slot = (d << 4) | lane
                            pos = plsc.load_gather(hists[j], [slot])
                            plsc.addupdate_scatter(hists[j], [slot], ones)
                            dst_store(p, j, pos, kk)
                    return 0
                lax.fori_loop(0, VR // 2, _perm, 0)

        def task_body(t, _):
            r0 = (wid * TPW + t) * G
            pltpu.sync_copy(v_hbm.at[pl.ds(r0, G), :], fbuf)
            for h in range(G // _NCOLS):
                process_group(r0, [h * _NCOLS + j for j in range(_NCOLS)])
            pltpu.sync_copy(fbuf, out_hbm.at[pl.ds(r0, G), :])
            return 0
        lax.fori_loop(0, TPW, task_body, 0)

    return run(vt, xors16)


def _bitonic_sort_body(sgn_ref, v_ref, o_ref):
    S = v_ref.shape[1]
    o_ref[0] = v_ref[0] * sgn_ref[0:1, :]
    iota = lax.broadcasted_iota(jnp.int32, (S, 1), 0)
    K = 2
    while K <= S:
        j = K // 2
        while j >= 1:
            x = o_ref[0]
            up = jnp.concatenate([x[j:], x[:j]], axis=0)
            down = jnp.concatenate([x[S - j:], x[:S - j]], axis=0)
            is_lo = (iota & j) == 0
            partner = jnp.where(is_lo, up, down)
            dir_asc = (iota & K) == 0
            want_min = is_lo == dir_asc
            o_ref[0] = jnp.where(want_min, jnp.minimum(x, partner),
                                 jnp.maximum(x, partner))
            j //= 2
        K *= 2
    o_ref[0] = o_ref[0] * sgn_ref[0:1, :]


def _tc_sort(vr, sgn):
    N, S, C = vr.shape
    return pl.pallas_call(
        _bitonic_sort_body,
        grid=(N,),
        in_specs=[
            pl.BlockSpec((8, C), lambda i: (0, 0)),
            pl.BlockSpec((1, S, C), lambda i: (i, 0, 0)),
        ],
        out_specs=pl.BlockSpec((1, S, C), lambda i: (i, 0, 0)),
        out_shape=jax.ShapeDtypeStruct((N, S, C), vr.dtype),
        compiler_params=pltpu.CompilerParams(
            vmem_limit_bytes=100 * 1024 * 1024),
    )(sgn, vr)


def kernel(q, k, v, col_descend):
    B, H, S, C = v.shape
    BH = B * H
    mask = jnp.zeros((C,), jnp.bool_).at[col_descend.reshape(-1)].set(True)
    xors = jnp.where(mask, jnp.int32(-1), jnp.int32(0))
    xors16 = jnp.broadcast_to(xors.reshape(C, 1), (C, 16)).reshape(C * 16)
    sgn = jnp.broadcast_to(
        jnp.where(mask, -1.0, 1.0).astype(v.dtype).reshape(1, C), (8, C))

    v3 = v.reshape(BH, S, C)
    NT = 28
    tc_out = _tc_sort(v3[:NT], sgn)
    vt = v3[NT:].swapaxes(1, 2).reshape((BH - NT) * C, S)
    sc_out = (_sc_sort_rows(vt, xors16, C)
              .reshape(BH - NT, C, S).swapaxes(1, 2))
    return jnp.concatenate([tc_out, sc_out], axis=0).reshape(B, H, S, C)
